# baseline (speedup 1.0000x reference)
import functools

import jax
import jax.numpy as jnp
from jax import lax
from jax.experimental import pallas as pl
from jax.experimental.pallas import tpu as pltpu
from jax.experimental.pallas import tpu_sc as plsc

VOCAB = 1_000_000
HIDDEN = 64
BATCH = 4096
HIST = 200

NUM_CORES = 2
NUM_SUBCORES = 16
NW = NUM_CORES * NUM_SUBCORES
ROWS_W = BATCH // NW

SPLIT = (128, 72)
NBUF = 4
LOOK = 2
N_GROUP = ROWS_W // NBUF

_mesh = plsc.VectorSubcoreMesh(core_axis_name="c", subcore_axis_name="s")


@functools.partial(
    pl.kernel,
    mesh=_mesh,
    compiler_params=pltpu.CompilerParams(use_tc_tiling_on_sc=False),
    out_type=jax.ShapeDtypeStruct((BATCH, HIST, HIDDEN), jnp.float32),
    scratch_types=[
        pltpu.VMEM((ROWS_W, HIST), jnp.int32),
        pltpu.VMEM((NBUF, HIST, HIDDEN), jnp.float32),
        pltpu.SemaphoreType.DMA((NBUF,)),
        pltpu.SemaphoreType.DMA((NBUF,)),
    ],
)
def _emb_lookup(ids_hbm, table_hbm, out_hbm, idx_v, rows_v, gsem, osem):
    wid = lax.axis_index("s") * NUM_CORES + lax.axis_index("c")
    base = wid * ROWS_W
    pltpu.sync_copy(ids_hbm.at[pl.ds(base, ROWS_W)], idx_v)

    def g_copies(j, b):
        off = 0
        copies = []
        for sz in SPLIT:
            copies.append(
                pltpu.make_async_copy(
                    table_hbm.at[idx_v.at[j, pl.ds(off, sz)]],
                    rows_v.at[b, pl.ds(off, sz)],
                    gsem.at[b],
                )
            )
            off += sz
        return copies

    def o_copy(j, b):
        return pltpu.make_async_copy(
            rows_v.at[b], out_hbm.at[base + j], osem.at[b]
        )

    def slot(g, b, head=False, tail=False):
        j = g * NBUF + b
        f = j + LOOK
        bf = (b + LOOK) % NBUF
        if not tail:
            if not head:
                o_copy(f - NBUF, bf).wait()
            for c in g_copies(f, bf):
                c.start()
        for c in g_copies(j, b):
            c.wait()
        o_copy(j, b).start()

    for b in range(LOOK):
        for c in g_copies(b, b):
            c.start()
    for b in range(NBUF):
        slot(0, b, head=b < NBUF - LOOK)

    def group_body(g, _):
        for b in range(NBUF):
            slot(g, b)
        return ()

    lax.fori_loop(1, N_GROUP - 1, group_body, (), unroll=False)

    for b in range(NBUF):
        slot(N_GROUP - 1, b, tail=b >= NBUF - LOOK)
    for b in range(NBUF):
        o_copy(ROWS_W - NBUF + b, b).wait()


def kernel(weight, input_ids):
    return _emb_lookup(input_ids.astype(jnp.int32), weight)
